# baseline (speedup 1.0000x reference)
import dataclasses
import functools

import jax
import jax.numpy as jnp
from jax import lax
from jax.experimental import pallas as pl
from jax.experimental.pallas import tpu as pltpu
from jax.experimental.pallas import tpu_sc as plsc

N = 10000
E = 25000
NODE_DIM = 1032
EDGE_DIM = 3096
H = 256

NC = 2
NS = 16
LANES = 16

EP = 25088
PW = EP // NS
CH = 112
NCH = PW // CH
NA = 10752
RZ = NA // NS

GW = EP // (NC * NS)
GNCH = GW // CH

NCNT = 10240

BM = 256
NB_N = 40
NB_A = NA // BM
NB_E = EP // BM

_EPS = 1e-5


def _ln(y, g, b):
    mu = jnp.mean(y, axis=-1, keepdims=True)
    var = jnp.mean((y - mu) ** 2, axis=-1, keepdims=True)
    return (y - mu) * lax.rsqrt(var + _EPS) * g + b


def _gelu(y):
    return 0.5 * y * (1.0 + lax.erf(y * (2.0 ** -0.5)))


def _node_enc_body(x_ref, wn_ref, bn_ref, gn_ref, betan_ref, wl_ref,
                   h_ref, hla_ref, hlb_ref):
    y = jnp.dot(x_ref[...], wn_ref[...], preferred_element_type=jnp.float32)
    h = _gelu(_ln(y + bn_ref[...], gn_ref[...], betan_ref[...]))
    h_ref[...] = h
    hl = jnp.dot(h, wl_ref[...], preferred_element_type=jnp.float32)
    hla_ref[...] = hl[:, :128]
    hlb_ref[...] = hl[:, 128:]


def _node_enc(x, wn, bn, gn, betan, wl):
    full = lambda shape: pl.BlockSpec(shape, lambda i: (0, 0))
    return pl.pallas_call(
        _node_enc_body,
        grid=(NB_N,),
        in_specs=[
            pl.BlockSpec((BM, NODE_DIM), lambda i: (i, 0)),
            full((NODE_DIM, H)), full((1, H)), full((1, H)), full((1, H)),
            full((H, H)),
        ],
        out_specs=[
            pl.BlockSpec((BM, H), lambda i: (i, 0)),
            pl.BlockSpec((BM, 128), lambda i: (i, 0)),
            pl.BlockSpec((BM, 128), lambda i: (i, 0)),
        ],
        out_shape=[
            jax.ShapeDtypeStruct((N, H), jnp.float32),
            jax.ShapeDtypeStruct((N, 128), jnp.float32),
            jax.ShapeDtypeStruct((N, 128), jnp.float32),
        ],
    )(x, wn, bn, gn, betan, wl)


def _edge_enc_body(ea_ref, we_ref, be_ref, ge_ref, betae_ref, w1c_ref, out_ref):
    y = jnp.dot(ea_ref[...].astype(jnp.bfloat16), we_ref[...],
                preferred_element_type=jnp.float32)
    eh = _gelu(_ln(y + be_ref[...], ge_ref[...], betae_ref[...]))
    out_ref[...] = jnp.dot(eh, w1c_ref[...], preferred_element_type=jnp.float32)


def _edge_enc(ea, we, be, ge, betae, w1c):
    full = lambda shape: pl.BlockSpec(shape, lambda i: (0, 0))
    return pl.pallas_call(
        _edge_enc_body,
        grid=(NB_E,),
        in_specs=[
            pl.BlockSpec((BM, EDGE_DIM), lambda i: (i, 0)),
            full((EDGE_DIM, H)), full((1, H)), full((1, H)), full((1, H)),
            full((H, H)),
        ],
        out_specs=pl.BlockSpec((BM, H), lambda i: (i, 0)),
        out_shape=jax.ShapeDtypeStruct((EP, H), jnp.float32),
    )(ea, we.astype(jnp.bfloat16), be, ge, betae, w1c)


def _upd_body(h_ref, sa_ref, sb_ref, cnt_ref, wr_ref, bl_ref, g_ref, beta_ref,
              wna_ref, wnb_ref, h_out, na_out, nb_out):
    s = jnp.concatenate([sa_ref[...], sb_ref[...]], axis=-1)
    deg = jnp.maximum(jnp.sum(cnt_ref[...], axis=0), 1.0)[:, None]
    y = s / deg + bl_ref[...] + jnp.dot(
        h_ref[...], wr_ref[...], preferred_element_type=jnp.float32)
    hn = _ln(_gelu(y), g_ref[...], beta_ref[...])
    h_out[...] = hn
    na_out[...] = jnp.dot(hn, wna_ref[...], preferred_element_type=jnp.float32)
    nb_out[...] = jnp.dot(hn, wnb_ref[...], preferred_element_type=jnp.float32)


def _upd(h, s2, cnt, wr, bl, g, beta, wna, wnb, nw):
    full = lambda shape: pl.BlockSpec(shape, lambda i: (0, 0))
    return pl.pallas_call(
        _upd_body,
        grid=(NB_N,),
        in_specs=[
            pl.BlockSpec((BM, H), lambda i: (i, 0)),
            pl.BlockSpec((BM, 128), lambda i: (i, 0)),
            pl.BlockSpec((BM, 128), lambda i: (NB_A + i, 0)),
            pl.BlockSpec((NC * NS, BM), lambda i: (0, i)),
            full((H, H)), full((1, H)), full((1, H)), full((1, H)),
            full((H, nw)), full((H, nw)),
        ],
        out_specs=[
            pl.BlockSpec((BM, H), lambda i: (i, 0)),
            pl.BlockSpec((BM, nw), lambda i: (i, 0)),
            pl.BlockSpec((BM, nw), lambda i: (i, 0)),
        ],
        out_shape=[
            jax.ShapeDtypeStruct((N, H), jnp.float32),
            jax.ShapeDtypeStruct((N, nw), jnp.float32),
            jax.ShapeDtypeStruct((N, nw), jnp.float32),
        ],
    )(h, s2, s2, cnt, wr, bl, g, beta, wna, wnb)


def _score_body(ga_ref, gb_ref, ec_ref, b1_ref, w2_ref, b2_ref, w3_ref,
                b3_ref, out_ref):
    z = _gelu(ga_ref[...] + gb_ref[...] + ec_ref[...] + b1_ref[...])
    z = _gelu(jnp.dot(z, w2_ref[...], preferred_element_type=jnp.float32)
              + b2_ref[...])
    out_ref[...] = jnp.dot(z, w3_ref[...],
                           preferred_element_type=jnp.float32) + b3_ref[...]


def _score(ga, gb, ec, b1, w2, b2, w3, b3):
    full = lambda shape: pl.BlockSpec(shape, lambda i: (0, 0))
    return pl.pallas_call(
        _score_body,
        grid=(NB_E,),
        in_specs=[
            pl.BlockSpec((BM, H), lambda i: (i, 0)),
            pl.BlockSpec((BM, H), lambda i: (i, 0)),
            pl.BlockSpec((BM, H), lambda i: (i, 0)),
            full((1, H)), full((H, H // 2)), full((1, H // 2)),
            full((H // 2, 1)), full((1, 1)),
        ],
        out_specs=pl.BlockSpec((BM, 1), lambda i: (i, 0)),
        out_shape=jax.ShapeDtypeStruct((EP, 1), jnp.float32),
    )(ga, gb, ec, b1, w2, b2, w3, b3)


_MESH = plsc.VectorSubcoreMesh(core_axis_name="c", subcore_axis_name="s")

_SC_PARAMS = pltpu.CompilerParams()
if "needs_layout_passes" in pltpu.CompilerParams.__dataclass_fields__:
    _SC_PARAMS = dataclasses.replace(_SC_PARAMS, needs_layout_passes=False)


def _faa_barrier(bar_smem, nonce, b):
    magic = jnp.int32(0x5C00000 + nonce * 64 + b * 2)
    bar_smem[2 * b] = magic

    def per_tile(t, carry):
        def cond(v):
            return v != magic

        def body(v):
            return plsc.fetch_and_add(bar_smem.at[2 * b], 0, subcore_id=t)

        lax.while_loop(cond, body, jnp.int32(0))
        return carry

    lax.fori_loop(0, NS, per_tile, jnp.int32(0))


def _make_segsum(nonce):
    @functools.partial(
        pl.kernel,
        mesh=_MESH,
        compiler_params=_SC_PARAMS,
        out_type=jax.ShapeDtypeStruct((2 * NA, 128), jnp.float32),
        scratch_types=[
            pltpu.VMEM((NCH, CH), jnp.int32),
            pltpu.VMEM((NCH, CH), jnp.int32),
            pltpu.VMEM((CH, 128), jnp.float32),
            pltpu.VMEM((RZ // CH, CH), jnp.int32),
            pltpu.VMEM_SHARED((NA, 128), jnp.float32),
            pltpu.SMEM((8,), jnp.int32),
        ],
    )
    def segsum(hl2_hbm, src2_hbm, dst_hbm, out_hbm,
               idxs_v, idxd_v, rows_v, idxl_v, acc_sh, bar_smem):
        c = lax.axis_index("c")
        s = lax.axis_index("s")
        base = s * PW
        zbase = s * RZ

        for kk in range(NCH):
            pltpu.sync_copy(
                src2_hbm.at[pl.ds(c * EP + base + kk * CH, CH)],
                idxs_v.at[kk])
            pltpu.sync_copy(dst_hbm.at[pl.ds(base + kk * CH, CH)],
                            idxd_v.at[kk])

        @pl.loop(0, CH)
        def _(i):
            @pl.loop(0, 128, step=LANES)
            def _(j):
                rows_v[i, pl.ds(j, LANES)] = jnp.zeros((LANES,), jnp.float32)

        for kk in range(RZ // CH):
            @pl.loop(0, CH, step=LANES)
            def _(j, kk=kk):
                idxl_v[kk, pl.ds(j, LANES)] = (
                    lax.iota(jnp.int32, LANES) + (zbase + kk * CH) + j)

        for kk in range(RZ // CH):
            pltpu.sync_copy(rows_v, acc_sh.at[idxl_v.at[kk]])

        _faa_barrier(bar_smem, nonce, 0)

        for kk in range(NCH):
            pltpu.sync_copy(hl2_hbm.at[idxs_v.at[kk]], rows_v)
            pltpu.sync_copy(rows_v, acc_sh.at[idxd_v.at[kk]], add=True)

        _faa_barrier(bar_smem, nonce, 1)

        for kk in range(RZ // CH):
            pltpu.sync_copy(
                acc_sh.at[pl.ds(zbase + kk * CH, CH)],
                out_hbm.at[pl.ds(c * NA + zbase + kk * CH, CH)])

    return segsum


_segsum1 = _make_segsum(1)
_segsum2 = _make_segsum(2)


@functools.partial(
    pl.kernel,
    mesh=_MESH,
    compiler_params=_SC_PARAMS,
    out_type=jax.ShapeDtypeStruct((NC * NS, NCNT), jnp.float32),
    scratch_types=[
        pltpu.VMEM((GNCH, CH), jnp.int32),
        pltpu.VMEM((NCNT,), jnp.float32),
    ],
)
def _degree_cnt(dst_hbm, out_hbm, idxd_v, acc_v):
    c = lax.axis_index("c")
    s = lax.axis_index("s")
    w = s * NC + c
    base = w * GW

    @pl.loop(0, NCNT, step=LANES)
    def _(i):
        acc_v[pl.ds(i, LANES)] = jnp.zeros((LANES,), jnp.float32)

    for kk in range(GNCH):
        pltpu.sync_copy(dst_hbm.at[pl.ds(base + kk * CH, CH)], idxd_v.at[kk])

    lanes_iota = lax.iota(jnp.int32, LANES)
    for kk in range(GNCH):
        @pl.loop(0, CH, step=LANES)
        def _(g, kk=kk):
            dvec = idxd_v[kk, pl.ds(g, LANES)]
            for i in range(LANES):
                d = dvec[i]
                b16 = (d // LANES) * LANES
                sel = (lanes_iota == (d - b16)).astype(jnp.float32)
                acc_v[pl.ds(b16, LANES)] = acc_v[pl.ds(b16, LANES)] + sel

    pltpu.sync_copy(acc_v, out_hbm.at[w])


@functools.partial(
    pl.kernel,
    mesh=_MESH,
    compiler_params=_SC_PARAMS,
    out_type=[
        jax.ShapeDtypeStruct((EP, H), jnp.float32),
        jax.ShapeDtypeStruct((EP, H), jnp.float32),
    ],
    scratch_types=[
        pltpu.VMEM((GNCH, CH), jnp.int32),
        pltpu.VMEM((GNCH, CH), jnp.int32),
        pltpu.VMEM((CH, H), jnp.float32),
    ],
)
def _pair_gather(ha_hbm, hb_hbm, src_hbm, dst_hbm, oa_hbm, ob_hbm,
                 idxs_v, idxd_v, buf_v):
    c = lax.axis_index("c")
    s = lax.axis_index("s")
    base = (s * NC + c) * GW

    for kk in range(GNCH):
        pltpu.sync_copy(src_hbm.at[pl.ds(base + kk * CH, CH)], idxs_v.at[kk])
        pltpu.sync_copy(dst_hbm.at[pl.ds(base + kk * CH, CH)], idxd_v.at[kk])

    for kk in range(GNCH):
        sl = pl.ds(base + kk * CH, CH)
        pltpu.sync_copy(ha_hbm.at[idxs_v.at[kk]], buf_v)
        pltpu.sync_copy(buf_v, oa_hbm.at[sl])
        pltpu.sync_copy(hb_hbm.at[idxd_v.at[kk]], buf_v)
        pltpu.sync_copy(buf_v, ob_hbm.at[sl])


def kernel(x, edge_index, edge_attr, Wn, bn, gn, betan, We, be, ge, betae,
           Wl0, Wr0, bl0, g0, beta0, Wl1, Wr1, bl1, g1, beta1,
           W1, b1, W2, b2, W3, b3):
    src = edge_index[0].astype(jnp.int32)
    dst = edge_index[1].astype(jnp.int32)
    pad = EP - E
    srcp = jnp.concatenate([src, jnp.zeros((pad,), jnp.int32)])
    src2 = jnp.concatenate([srcp, srcp + N])
    dstp_seg = jnp.concatenate(
        [dst, N + (jnp.arange(pad, dtype=jnp.int32) % 16)])
    dstp_g = jnp.concatenate([dst, jnp.zeros((pad,), jnp.int32)])

    row = lambda v: v.reshape(1, -1)
    w1a, w1b, w1c = W1[:H], W1[H:2 * H], W1[2 * H:]

    cnt = _degree_cnt(dstp_seg)
    ehc = _edge_enc(edge_attr, We, row(be), row(ge), row(betae), w1c)

    h0, hl0a, hl0b = _node_enc(x, Wn, row(bn), row(gn), row(betan), Wl0)
    s20 = _segsum1(jnp.concatenate([hl0a, hl0b], axis=0), src2, dstp_seg)
    h1, hl1a, hl1b = _upd(h0, s20, cnt, Wr0, row(bl0), row(g0), row(beta0),
                          Wl1[:, :128], Wl1[:, 128:], 128)
    s21 = _segsum2(jnp.concatenate([hl1a, hl1b], axis=0), src2, dstp_seg)
    h2, ha, hb = _upd(h1, s21, cnt, Wr1, row(bl1), row(g1), row(beta1),
                      w1a, w1b, H)

    ga, gb = _pair_gather(ha, hb, srcp, dstp_g)
    out = _score(ga, gb, ehc, row(b1), W2, row(b2), W3, row(b3))
    return out[:E, 0]


# speedup vs baseline: 1.0226x; 1.0226x over previous
import dataclasses
import functools

import jax
import jax.numpy as jnp
from jax import lax
from jax.experimental import pallas as pl
from jax.experimental.pallas import tpu as pltpu
from jax.experimental.pallas import tpu_sc as plsc

N = 10000
E = 25000
NODE_DIM = 1032
EDGE_DIM = 3096
H = 256

NC = 2
NS = 16
LANES = 16

EP = 25088
PW = EP // NS
CH = 112
NCH = PW // CH
NA = 10752
RZ = NA // NS

GW = EP // (NC * NS)
GNCH = GW // CH

NCNT = 10240

BM = 256
NB_N = 40
NB_A = NA // BM
NB_E = EP // BM

_EPS = 1e-5


def _ln(y, g, b):
    mu = jnp.mean(y, axis=-1, keepdims=True)
    var = jnp.mean((y - mu) ** 2, axis=-1, keepdims=True)
    return (y - mu) * lax.rsqrt(var + _EPS) * g + b


def _gelu(y):
    return 0.5 * y * (1.0 + lax.erf(y * (2.0 ** -0.5)))


def _node_enc_body(x_ref, wn_ref, bn_ref, gn_ref, betan_ref, wl_ref,
                   h_ref, hla_ref, hlb_ref):
    y = jnp.dot(x_ref[...], wn_ref[...], preferred_element_type=jnp.float32)
    h = _gelu(_ln(y + bn_ref[...], gn_ref[...], betan_ref[...]))
    h_ref[...] = h
    hl = jnp.dot(h, wl_ref[...], preferred_element_type=jnp.float32)
    hla_ref[...] = hl[:, :128]
    hlb_ref[...] = hl[:, 128:]


def _node_enc(x, wn, bn, gn, betan, wl):
    full = lambda shape: pl.BlockSpec(shape, lambda i: (0, 0))
    return pl.pallas_call(
        _node_enc_body,
        grid=(NB_N,),
        in_specs=[
            pl.BlockSpec((BM, NODE_DIM), lambda i: (i, 0)),
            full((NODE_DIM, H)), full((1, H)), full((1, H)), full((1, H)),
            full((H, H)),
        ],
        out_specs=[
            pl.BlockSpec((BM, H), lambda i: (i, 0)),
            pl.BlockSpec((BM, 128), lambda i: (i, 0)),
            pl.BlockSpec((BM, 128), lambda i: (i, 0)),
        ],
        out_shape=[
            jax.ShapeDtypeStruct((N, H), jnp.float32),
            jax.ShapeDtypeStruct((N, 128), jnp.float32),
            jax.ShapeDtypeStruct((N, 128), jnp.float32),
        ],
    )(x, wn, bn, gn, betan, wl)


def _edge_enc_body(ea_ref, we_ref, be_ref, ge_ref, betae_ref, w1c_ref, out_ref):
    y = jnp.dot(ea_ref[...].astype(jnp.bfloat16), we_ref[...],
                preferred_element_type=jnp.float32)
    eh = _gelu(_ln(y + be_ref[...], ge_ref[...], betae_ref[...]))
    out_ref[...] = jnp.dot(eh, w1c_ref[...], preferred_element_type=jnp.float32)


def _edge_enc(ea, we, be, ge, betae, w1c):
    full = lambda shape: pl.BlockSpec(shape, lambda i: (0, 0))
    return pl.pallas_call(
        _edge_enc_body,
        grid=(NB_E,),
        in_specs=[
            pl.BlockSpec((BM, EDGE_DIM), lambda i: (i, 0)),
            full((EDGE_DIM, H)), full((1, H)), full((1, H)), full((1, H)),
            full((H, H)),
        ],
        out_specs=pl.BlockSpec((BM, H), lambda i: (i, 0)),
        out_shape=jax.ShapeDtypeStruct((EP, H), jnp.float32),
    )(ea, we.astype(jnp.bfloat16), be, ge, betae, w1c)


def _upd_body(h_ref, sa_ref, sb_ref, cnt_ref, wr_ref, bl_ref, g_ref, beta_ref,
              wna_ref, wnb_ref, h_out, na_out, nb_out):
    s = jnp.concatenate([sa_ref[...], sb_ref[...]], axis=-1)
    deg = jnp.maximum(jnp.sum(cnt_ref[...], axis=0), 1.0)[:, None]
    y = s / deg + bl_ref[...] + jnp.dot(
        h_ref[...], wr_ref[...], preferred_element_type=jnp.float32)
    hn = _ln(_gelu(y), g_ref[...], beta_ref[...])
    h_out[...] = hn
    na_out[...] = jnp.dot(hn, wna_ref[...], preferred_element_type=jnp.float32)
    nb_out[...] = jnp.dot(hn, wnb_ref[...], preferred_element_type=jnp.float32)


def _upd(h, s2, cnt, wr, bl, g, beta, wna, wnb, nw):
    full = lambda shape: pl.BlockSpec(shape, lambda i: (0, 0))
    return pl.pallas_call(
        _upd_body,
        grid=(NB_N,),
        in_specs=[
            pl.BlockSpec((BM, H), lambda i: (i, 0)),
            pl.BlockSpec((BM, 128), lambda i: (i, 0)),
            pl.BlockSpec((BM, 128), lambda i: (NB_A + i, 0)),
            pl.BlockSpec((NC * NS, BM), lambda i: (0, i)),
            full((H, H)), full((1, H)), full((1, H)), full((1, H)),
            full((H, nw)), full((H, nw)),
        ],
        out_specs=[
            pl.BlockSpec((BM, H), lambda i: (i, 0)),
            pl.BlockSpec((BM, nw), lambda i: (i, 0)),
            pl.BlockSpec((BM, nw), lambda i: (i, 0)),
        ],
        out_shape=[
            jax.ShapeDtypeStruct((N, H), jnp.float32),
            jax.ShapeDtypeStruct((N, nw), jnp.float32),
            jax.ShapeDtypeStruct((N, nw), jnp.float32),
        ],
    )(h, s2, s2, cnt, wr, bl, g, beta, wna, wnb)


def _score_body(ga_ref, gb_ref, ec_ref, b1_ref, w2_ref, b2_ref, w3_ref,
                b3_ref, out_ref):
    z = _gelu(ga_ref[...] + gb_ref[...] + ec_ref[...] + b1_ref[...])
    z = _gelu(jnp.dot(z, w2_ref[...], preferred_element_type=jnp.float32)
              + b2_ref[...])
    out_ref[...] = jnp.dot(z, w3_ref[...],
                           preferred_element_type=jnp.float32) + b3_ref[...]


def _score(ga, gb, ec, b1, w2, b2, w3, b3):
    full = lambda shape: pl.BlockSpec(shape, lambda i: (0, 0))
    return pl.pallas_call(
        _score_body,
        grid=(NB_E,),
        in_specs=[
            pl.BlockSpec((BM, H), lambda i: (i, 0)),
            pl.BlockSpec((BM, H), lambda i: (i, 0)),
            pl.BlockSpec((BM, H), lambda i: (i, 0)),
            full((1, H)), full((H, H // 2)), full((1, H // 2)),
            full((H // 2, 1)), full((1, 1)),
        ],
        out_specs=pl.BlockSpec((BM, 1), lambda i: (i, 0)),
        out_shape=jax.ShapeDtypeStruct((EP, 1), jnp.float32),
    )(ga, gb, ec, b1, w2, b2, w3, b3)


_MESH = plsc.VectorSubcoreMesh(core_axis_name="c", subcore_axis_name="s")

_SC_PARAMS = pltpu.CompilerParams()
if "needs_layout_passes" in pltpu.CompilerParams.__dataclass_fields__:
    _SC_PARAMS = dataclasses.replace(_SC_PARAMS, needs_layout_passes=False)


def _faa_barrier(bar_smem, nonce, b):
    magic = jnp.int32(0x5C00000 + nonce * 64 + b * 2)
    bar_smem[2 * b] = magic

    def per_tile(t, carry):
        def cond(v):
            return v != magic

        def body(v):
            return plsc.fetch_and_add(bar_smem.at[2 * b], 0, subcore_id=t)

        lax.while_loop(cond, body, jnp.int32(0))
        return carry

    lax.fori_loop(0, NS, per_tile, jnp.int32(0))


def _make_segsum(nonce):
    @functools.partial(
        pl.kernel,
        mesh=_MESH,
        compiler_params=_SC_PARAMS,
        out_type=jax.ShapeDtypeStruct((2 * NA, 128), jnp.float32),
        scratch_types=[
            pltpu.VMEM((NCH, CH), jnp.int32),
            pltpu.VMEM((NCH, CH), jnp.int32),
            pltpu.VMEM((CH, 128), jnp.float32),
            pltpu.VMEM((CH, 128), jnp.float32),
            pltpu.VMEM((RZ // CH, CH), jnp.int32),
            pltpu.VMEM_SHARED((NA, 128), jnp.float32),
            pltpu.SMEM((8,), jnp.int32),
            pltpu.SemaphoreType.DMA,
            pltpu.SemaphoreType.DMA,
        ],
    )
    def segsum(hl2_hbm, src2_hbm, dst_hbm, out_hbm,
               idxs_v, idxd_v, rows_v, rows2_v, idxl_v, acc_sh, bar_smem,
               sem0, sem1):
        c = lax.axis_index("c")
        s = lax.axis_index("s")
        base = s * PW
        zbase = s * RZ

        for kk in range(NCH):
            pltpu.sync_copy(
                src2_hbm.at[pl.ds(c * EP + base + kk * CH, CH)],
                idxs_v.at[kk])
            pltpu.sync_copy(dst_hbm.at[pl.ds(base + kk * CH, CH)],
                            idxd_v.at[kk])

        @pl.loop(0, CH)
        def _(i):
            @pl.loop(0, 128, step=LANES)
            def _(j):
                rows_v[i, pl.ds(j, LANES)] = jnp.zeros((LANES,), jnp.float32)

        for kk in range(RZ // CH):
            @pl.loop(0, CH, step=LANES)
            def _(j, kk=kk):
                idxl_v[kk, pl.ds(j, LANES)] = (
                    lax.iota(jnp.int32, LANES) + (zbase + kk * CH) + j)

        for kk in range(RZ // CH):
            pltpu.sync_copy(rows_v, acc_sh.at[idxl_v.at[kk]])

        _faa_barrier(bar_smem, nonce, 0)

        bufs = (rows_v, rows2_v)
        sems = (sem0, sem1)
        handles = [None] * NCH
        handles[0] = pltpu.async_copy(hl2_hbm.at[idxs_v.at[0]], bufs[0],
                                      sems[0])
        for kk in range(NCH):
            if kk + 1 < NCH:
                handles[kk + 1] = pltpu.async_copy(
                    hl2_hbm.at[idxs_v.at[kk + 1]], bufs[(kk + 1) % 2],
                    sems[(kk + 1) % 2])
            handles[kk].wait()
            pltpu.sync_copy(bufs[kk % 2], acc_sh.at[idxd_v.at[kk]], add=True)

        _faa_barrier(bar_smem, nonce, 1)

        for kk in range(RZ // CH):
            pltpu.sync_copy(
                acc_sh.at[pl.ds(zbase + kk * CH, CH)],
                out_hbm.at[pl.ds(c * NA + zbase + kk * CH, CH)])

    return segsum


_segsum1 = _make_segsum(1)
_segsum2 = _make_segsum(2)


@functools.partial(
    pl.kernel,
    mesh=_MESH,
    compiler_params=_SC_PARAMS,
    out_type=jax.ShapeDtypeStruct((NC * NS, NCNT), jnp.float32),
    scratch_types=[
        pltpu.VMEM((GNCH, CH), jnp.int32),
        pltpu.VMEM((NCNT,), jnp.float32),
    ],
)
def _degree_cnt(dst_hbm, out_hbm, idxd_v, acc_v):
    c = lax.axis_index("c")
    s = lax.axis_index("s")
    w = s * NC + c
    base = w * GW

    @pl.loop(0, NCNT, step=LANES)
    def _(i):
        acc_v[pl.ds(i, LANES)] = jnp.zeros((LANES,), jnp.float32)

    for kk in range(GNCH):
        pltpu.sync_copy(dst_hbm.at[pl.ds(base + kk * CH, CH)], idxd_v.at[kk])

    lanes_iota = lax.iota(jnp.int32, LANES)
    for kk in range(GNCH):
        @pl.loop(0, CH, step=LANES)
        def _(g, kk=kk):
            dvec = idxd_v[kk, pl.ds(g, LANES)]
            for i in range(LANES):
                d = dvec[i]
                b16 = (d // LANES) * LANES
                sel = (lanes_iota == (d - b16)).astype(jnp.float32)
                acc_v[pl.ds(b16, LANES)] = acc_v[pl.ds(b16, LANES)] + sel

    pltpu.sync_copy(acc_v, out_hbm.at[w])


@functools.partial(
    pl.kernel,
    mesh=_MESH,
    compiler_params=_SC_PARAMS,
    out_type=[
        jax.ShapeDtypeStruct((EP, H), jnp.float32),
        jax.ShapeDtypeStruct((EP, H), jnp.float32),
    ],
    scratch_types=[
        pltpu.VMEM((GNCH, CH), jnp.int32),
        pltpu.VMEM((GNCH, CH), jnp.int32),
        pltpu.VMEM((CH, H), jnp.float32),
        pltpu.VMEM((CH, H), jnp.float32),
        pltpu.VMEM((CH, H), jnp.float32),
        pltpu.VMEM((CH, H), jnp.float32),
        pltpu.SemaphoreType.DMA,
        pltpu.SemaphoreType.DMA,
        pltpu.SemaphoreType.DMA,
        pltpu.SemaphoreType.DMA,
    ],
)
def _pair_gather(ha_hbm, hb_hbm, src_hbm, dst_hbm, oa_hbm, ob_hbm,
                 idxs_v, idxd_v, ba0, ba1, bb0, bb1, sa0, sa1, sb0, sb1):
    c = lax.axis_index("c")
    s = lax.axis_index("s")
    base = (s * NC + c) * GW

    for kk in range(GNCH):
        pltpu.sync_copy(src_hbm.at[pl.ds(base + kk * CH, CH)], idxs_v.at[kk])
        pltpu.sync_copy(dst_hbm.at[pl.ds(base + kk * CH, CH)], idxd_v.at[kk])

    bufa = (ba0, ba1)
    bufb = (bb0, bb1)
    sema = (sa0, sa1)
    semb = (sb0, sb1)

    def fire(kk):
        p = kk % 2
        ha_h = pltpu.async_copy(ha_hbm.at[idxs_v.at[kk]], bufa[p], sema[p])
        hb_h = pltpu.async_copy(hb_hbm.at[idxd_v.at[kk]], bufb[p], semb[p])
        return ha_h, hb_h

    handles = [None] * GNCH
    handles[0] = fire(0)
    for kk in range(GNCH):
        if kk + 1 < GNCH:
            handles[kk + 1] = fire(kk + 1)
        sl = pl.ds(base + kk * CH, CH)
        p = kk % 2
        ha_h, hb_h = handles[kk]
        ha_h.wait()
        pltpu.sync_copy(bufa[p], oa_hbm.at[sl])
        hb_h.wait()
        pltpu.sync_copy(bufb[p], ob_hbm.at[sl])


def kernel(x, edge_index, edge_attr, Wn, bn, gn, betan, We, be, ge, betae,
           Wl0, Wr0, bl0, g0, beta0, Wl1, Wr1, bl1, g1, beta1,
           W1, b1, W2, b2, W3, b3):
    src = edge_index[0].astype(jnp.int32)
    dst = edge_index[1].astype(jnp.int32)
    pad = EP - E
    srcp = jnp.concatenate([src, jnp.zeros((pad,), jnp.int32)])
    src2 = jnp.concatenate([srcp, srcp + N])
    dstp_seg = jnp.concatenate(
        [dst, N + (jnp.arange(pad, dtype=jnp.int32) % 16)])
    dstp_g = jnp.concatenate([dst, jnp.zeros((pad,), jnp.int32)])

    row = lambda v: v.reshape(1, -1)
    w1a, w1b, w1c = W1[:H], W1[H:2 * H], W1[2 * H:]

    cnt = _degree_cnt(dstp_seg)
    ehc = _edge_enc(edge_attr, We, row(be), row(ge), row(betae), w1c)

    h0, hl0a, hl0b = _node_enc(x, Wn, row(bn), row(gn), row(betan), Wl0)
    s20 = _segsum1(jnp.concatenate([hl0a, hl0b], axis=0), src2, dstp_seg)
    h1, hl1a, hl1b = _upd(h0, s20, cnt, Wr0, row(bl0), row(g0), row(beta0),
                          Wl1[:, :128], Wl1[:, 128:], 128)
    s21 = _segsum2(jnp.concatenate([hl1a, hl1b], axis=0), src2, dstp_seg)
    h2, ha, hb = _upd(h1, s21, cnt, Wr1, row(bl1), row(g1), row(beta1),
                      w1a, w1b, H)

    ga, gb = _pair_gather(ha, hb, srcp, dstp_g)
    out = _score(ga, gb, ehc, row(b1), W2, row(b2), W3, row(b3))
    return out[:E, 0]


# speedup vs baseline: 1.0420x; 1.0191x over previous
import dataclasses
import functools

import jax
import jax.numpy as jnp
from jax import lax
from jax.experimental import pallas as pl
from jax.experimental.pallas import tpu as pltpu
from jax.experimental.pallas import tpu_sc as plsc

N = 10000
E = 25000
NODE_DIM = 1032
EDGE_DIM = 3096
H = 256

NC = 2
NS = 16
LANES = 16

EP = 25088
PW = EP // NS
CH = 112
NCH = PW // CH
NA = 10752
RZ = NA // NS

GW = EP // (NC * NS)
GNCH = GW // CH

NCNT = 10240

BM = 256
NB_N = 40
NB_A = NA // BM
NB_E = EP // BM

_EPS = 1e-5


def _ln(y, g, b):
    mu = jnp.mean(y, axis=-1, keepdims=True)
    var = jnp.mean((y - mu) ** 2, axis=-1, keepdims=True)
    return (y - mu) * lax.rsqrt(var + _EPS) * g + b


def _gelu(y):
    return 0.5 * y * (1.0 + lax.erf(y * (2.0 ** -0.5)))


def _node_enc_body(x_ref, wn_ref, bn_ref, gn_ref, betan_ref, wl_ref,
                   h_ref, hl_ref):
    y = jnp.dot(x_ref[...], wn_ref[...], preferred_element_type=jnp.float32)
    h = _gelu(_ln(y + bn_ref[...], gn_ref[...], betan_ref[...]))
    h_ref[...] = h
    hl = jnp.dot(h, wl_ref[...], preferred_element_type=jnp.float32)
    hl_ref[:, 0, :] = hl[:, :128]
    hl_ref[:, 1, :] = hl[:, 128:]


def _node_enc(x, wn, bn, gn, betan, wl):
    full = lambda shape: pl.BlockSpec(shape, lambda i: (0, 0))
    return pl.pallas_call(
        _node_enc_body,
        grid=(NB_N,),
        in_specs=[
            pl.BlockSpec((BM, NODE_DIM), lambda i: (i, 0)),
            full((NODE_DIM, H)), full((1, H)), full((1, H)), full((1, H)),
            full((H, H)),
        ],
        out_specs=[
            pl.BlockSpec((BM, H), lambda i: (i, 0)),
            pl.BlockSpec((BM, 2, 128), lambda i: (i, 0, 0)),
        ],
        out_shape=[
            jax.ShapeDtypeStruct((N, H), jnp.float32),
            jax.ShapeDtypeStruct((N, 2, 128), jnp.float32),
        ],
    )(x, wn, bn, gn, betan, wl)


def _edge_enc_body(ea_ref, we_ref, be_ref, ge_ref, betae_ref, w1c_ref, out_ref):
    y = jnp.dot(ea_ref[...].astype(jnp.bfloat16), we_ref[...],
                preferred_element_type=jnp.float32)
    eh = _gelu(_ln(y + be_ref[...], ge_ref[...], betae_ref[...]))
    out_ref[...] = jnp.dot(eh, w1c_ref[...], preferred_element_type=jnp.float32)


def _edge_enc(ea, we, be, ge, betae, w1c):
    full = lambda shape: pl.BlockSpec(shape, lambda i: (0, 0))
    return pl.pallas_call(
        _edge_enc_body,
        grid=(NB_E,),
        in_specs=[
            pl.BlockSpec((BM, EDGE_DIM), lambda i: (i, 0)),
            full((EDGE_DIM, H)), full((1, H)), full((1, H)), full((1, H)),
            full((H, H)),
        ],
        out_specs=pl.BlockSpec((BM, H), lambda i: (i, 0)),
        out_shape=jax.ShapeDtypeStruct((EP, H), jnp.float32),
    )(ea, we.astype(jnp.bfloat16), be, ge, betae, w1c)


def _upd_body(inter, h_ref, sa_ref, sb_ref, cnt_ref, wr_ref, bl_ref, g_ref,
              beta_ref, wna_ref, wnb_ref, h_out, *outs):
    s = jnp.concatenate([sa_ref[...], sb_ref[...]], axis=-1)
    deg = jnp.maximum(jnp.sum(cnt_ref[...], axis=0), 1.0)[:, None]
    y = s / deg + bl_ref[...] + jnp.dot(
        h_ref[...], wr_ref[...], preferred_element_type=jnp.float32)
    hn = _ln(_gelu(y), g_ref[...], beta_ref[...])
    h_out[...] = hn
    na = jnp.dot(hn, wna_ref[...], preferred_element_type=jnp.float32)
    nb = jnp.dot(hn, wnb_ref[...], preferred_element_type=jnp.float32)
    if inter:
        outs[0][:, 0, :] = na
        outs[0][:, 1, :] = nb
    else:
        outs[0][...] = na
        outs[1][...] = nb


def _upd(h, s2, cnt, wr, bl, g, beta, wna, wnb, inter):
    full = lambda shape: pl.BlockSpec(shape, lambda i: (0, 0))
    nw = 128 if inter else H
    if inter:
        out_specs = [
            pl.BlockSpec((BM, H), lambda i: (i, 0)),
            pl.BlockSpec((BM, 2, 128), lambda i: (i, 0, 0)),
        ]
        out_shape = [
            jax.ShapeDtypeStruct((N, H), jnp.float32),
            jax.ShapeDtypeStruct((N, 2, 128), jnp.float32),
        ]
    else:
        out_specs = [
            pl.BlockSpec((BM, H), lambda i: (i, 0)),
            pl.BlockSpec((BM, nw), lambda i: (i, 0)),
            pl.BlockSpec((BM, nw), lambda i: (i, 0)),
        ]
        out_shape = [
            jax.ShapeDtypeStruct((N, H), jnp.float32),
            jax.ShapeDtypeStruct((N, nw), jnp.float32),
            jax.ShapeDtypeStruct((N, nw), jnp.float32),
        ]
    return pl.pallas_call(
        functools.partial(_upd_body, inter),
        grid=(NB_N,),
        in_specs=[
            pl.BlockSpec((BM, H), lambda i: (i, 0)),
            pl.BlockSpec((BM, 128), lambda i: (i, 0)),
            pl.BlockSpec((BM, 128), lambda i: (NB_A + i, 0)),
            pl.BlockSpec((NC * NS, BM), lambda i: (0, i)),
            full((H, H)), full((1, H)), full((1, H)), full((1, H)),
            full((H, nw)), full((H, nw)),
        ],
        out_specs=out_specs,
        out_shape=out_shape,
    )(h, s2, s2, cnt, wr, bl, g, beta, wna, wnb)


def _score_body(ga_ref, gb_ref, ec_ref, b1_ref, w2_ref, b2_ref, w3_ref,
                b3_ref, out_ref):
    z = _gelu(ga_ref[...] + gb_ref[...] + ec_ref[...] + b1_ref[...])
    z = _gelu(jnp.dot(z, w2_ref[...], preferred_element_type=jnp.float32)
              + b2_ref[...])
    out_ref[...] = jnp.dot(z, w3_ref[...],
                           preferred_element_type=jnp.float32) + b3_ref[...]


def _score(ga, gb, ec, b1, w2, b2, w3, b3):
    full = lambda shape: pl.BlockSpec(shape, lambda i: (0, 0))
    return pl.pallas_call(
        _score_body,
        grid=(NB_E,),
        in_specs=[
            pl.BlockSpec((BM, H), lambda i: (i, 0)),
            pl.BlockSpec((BM, H), lambda i: (i, 0)),
            pl.BlockSpec((BM, H), lambda i: (i, 0)),
            full((1, H)), full((H, H // 2)), full((1, H // 2)),
            full((H // 2, 1)), full((1, 1)),
        ],
        out_specs=pl.BlockSpec((BM, 1), lambda i: (i, 0)),
        out_shape=jax.ShapeDtypeStruct((EP, 1), jnp.float32),
    )(ga, gb, ec, b1, w2, b2, w3, b3)


_MESH = plsc.VectorSubcoreMesh(core_axis_name="c", subcore_axis_name="s")

_SC_PARAMS = pltpu.CompilerParams()
if "needs_layout_passes" in pltpu.CompilerParams.__dataclass_fields__:
    _SC_PARAMS = dataclasses.replace(_SC_PARAMS, needs_layout_passes=False)


def _faa_barrier(bar_smem, nonce, b):
    magic = jnp.int32(0x5C00000 + nonce * 64 + b * 2)
    bar_smem[2 * b] = magic

    def per_tile(t, carry):
        def cond(v):
            return v != magic

        def body(v):
            return plsc.fetch_and_add(bar_smem.at[2 * b], 0, subcore_id=t)

        lax.while_loop(cond, body, jnp.int32(0))
        return carry

    lax.fori_loop(0, NS, per_tile, jnp.int32(0))


def _make_segsum(nonce):
    @functools.partial(
        pl.kernel,
        mesh=_MESH,
        compiler_params=_SC_PARAMS,
        out_type=jax.ShapeDtypeStruct((2 * NA, 128), jnp.float32),
        scratch_types=[
            pltpu.VMEM((NCH, CH), jnp.int32),
            pltpu.VMEM((NCH, CH), jnp.int32),
            pltpu.VMEM((CH, 128), jnp.float32),
            pltpu.VMEM((CH, 128), jnp.float32),
            pltpu.VMEM((RZ // CH, CH), jnp.int32),
            pltpu.VMEM_SHARED((NA, 128), jnp.float32),
            pltpu.SMEM((8,), jnp.int32),
            pltpu.SemaphoreType.DMA,
            pltpu.SemaphoreType.DMA,
        ],
    )
    def segsum(hl2_hbm, src2_hbm, dst_hbm, out_hbm,
               idxs_v, idxd_v, rows_v, rows2_v, idxl_v, acc_sh, bar_smem,
               sem0, sem1):
        c = lax.axis_index("c")
        s = lax.axis_index("s")
        base = s * PW
        zbase = s * RZ

        for kk in range(NCH):
            pltpu.sync_copy(
                src2_hbm.at[pl.ds(c * EP + base + kk * CH, CH)],
                idxs_v.at[kk])
            pltpu.sync_copy(dst_hbm.at[pl.ds(base + kk * CH, CH)],
                            idxd_v.at[kk])

        @pl.loop(0, CH)
        def _(i):
            @pl.loop(0, 128, step=LANES)
            def _(j):
                rows_v[i, pl.ds(j, LANES)] = jnp.zeros((LANES,), jnp.float32)

        for kk in range(RZ // CH):
            @pl.loop(0, CH, step=LANES)
            def _(j, kk=kk):
                idxl_v[kk, pl.ds(j, LANES)] = (
                    lax.iota(jnp.int32, LANES) + (zbase + kk * CH) + j)

        for kk in range(RZ // CH):
            pltpu.sync_copy(rows_v, acc_sh.at[idxl_v.at[kk]])

        _faa_barrier(bar_smem, nonce, 0)

        bufs = (rows_v, rows2_v)
        sems = (sem0, sem1)
        handles = [None] * NCH
        handles[0] = pltpu.async_copy(hl2_hbm.at[idxs_v.at[0]], bufs[0],
                                      sems[0])
        for kk in range(NCH):
            if kk + 1 < NCH:
                handles[kk + 1] = pltpu.async_copy(
                    hl2_hbm.at[idxs_v.at[kk + 1]], bufs[(kk + 1) % 2],
                    sems[(kk + 1) % 2])
            handles[kk].wait()
            pltpu.sync_copy(bufs[kk % 2], acc_sh.at[idxd_v.at[kk]], add=True)

        _faa_barrier(bar_smem, nonce, 1)

        for kk in range(RZ // CH):
            pltpu.sync_copy(
                acc_sh.at[pl.ds(zbase + kk * CH, CH)],
                out_hbm.at[pl.ds(c * NA + zbase + kk * CH, CH)])

    return segsum


_segsum1 = _make_segsum(1)
_segsum2 = _make_segsum(2)


@functools.partial(
    pl.kernel,
    mesh=_MESH,
    compiler_params=_SC_PARAMS,
    out_type=jax.ShapeDtypeStruct((NC * NS, NCNT), jnp.float32),
    scratch_types=[
        pltpu.VMEM((GNCH, CH), jnp.int32),
        pltpu.VMEM((NCNT,), jnp.float32),
    ],
)
def _degree_cnt(dst_hbm, out_hbm, idxd_v, acc_v):
    c = lax.axis_index("c")
    s = lax.axis_index("s")
    w = s * NC + c
    base = w * GW

    @pl.loop(0, NCNT, step=LANES)
    def _(i):
        acc_v[pl.ds(i, LANES)] = jnp.zeros((LANES,), jnp.float32)

    for kk in range(GNCH):
        pltpu.sync_copy(dst_hbm.at[pl.ds(base + kk * CH, CH)], idxd_v.at[kk])

    lanes_iota = lax.iota(jnp.int32, LANES)
    for kk in range(GNCH):
        @pl.loop(0, CH, step=LANES)
        def _(g, kk=kk):
            dvec = idxd_v[kk, pl.ds(g, LANES)]
            for i in range(LANES):
                d = dvec[i]
                b16 = (d // LANES) * LANES
                sel = (lanes_iota == (d - b16)).astype(jnp.float32)
                acc_v[pl.ds(b16, LANES)] = acc_v[pl.ds(b16, LANES)] + sel

    pltpu.sync_copy(acc_v, out_hbm.at[w])


@functools.partial(
    pl.kernel,
    mesh=_MESH,
    compiler_params=_SC_PARAMS,
    out_type=[
        jax.ShapeDtypeStruct((EP, H), jnp.float32),
        jax.ShapeDtypeStruct((EP, H), jnp.float32),
    ],
    scratch_types=[
        pltpu.VMEM((GNCH, CH), jnp.int32),
        pltpu.VMEM((GNCH, CH), jnp.int32),
        pltpu.VMEM((CH, H), jnp.float32),
        pltpu.VMEM((CH, H), jnp.float32),
        pltpu.VMEM((CH, H), jnp.float32),
        pltpu.VMEM((CH, H), jnp.float32),
        pltpu.SemaphoreType.DMA,
        pltpu.SemaphoreType.DMA,
        pltpu.SemaphoreType.DMA,
        pltpu.SemaphoreType.DMA,
    ],
)
def _pair_gather(ha_hbm, hb_hbm, src_hbm, dst_hbm, oa_hbm, ob_hbm,
                 idxs_v, idxd_v, ba0, ba1, bb0, bb1, sa0, sa1, sb0, sb1):
    c = lax.axis_index("c")
    s = lax.axis_index("s")
    base = (s * NC + c) * GW

    for kk in range(GNCH):
        pltpu.sync_copy(src_hbm.at[pl.ds(base + kk * CH, CH)], idxs_v.at[kk])
        pltpu.sync_copy(dst_hbm.at[pl.ds(base + kk * CH, CH)], idxd_v.at[kk])

    bufa = (ba0, ba1)
    bufb = (bb0, bb1)
    sema = (sa0, sa1)
    semb = (sb0, sb1)

    def fire(kk):
        p = kk % 2
        ha_h = pltpu.async_copy(ha_hbm.at[idxs_v.at[kk]], bufa[p], sema[p])
        hb_h = pltpu.async_copy(hb_hbm.at[idxd_v.at[kk]], bufb[p], semb[p])
        return ha_h, hb_h

    handles = [None] * GNCH
    handles[0] = fire(0)
    for kk in range(GNCH):
        if kk + 1 < GNCH:
            handles[kk + 1] = fire(kk + 1)
        sl = pl.ds(base + kk * CH, CH)
        p = kk % 2
        ha_h, hb_h = handles[kk]
        ha_h.wait()
        pltpu.sync_copy(bufa[p], oa_hbm.at[sl])
        hb_h.wait()
        pltpu.sync_copy(bufb[p], ob_hbm.at[sl])


def kernel(x, edge_index, edge_attr, Wn, bn, gn, betan, We, be, ge, betae,
           Wl0, Wr0, bl0, g0, beta0, Wl1, Wr1, bl1, g1, beta1,
           W1, b1, W2, b2, W3, b3):
    src = edge_index[0].astype(jnp.int32)
    dst = edge_index[1].astype(jnp.int32)
    pad = EP - E
    srcp = jnp.concatenate([src, jnp.zeros((pad,), jnp.int32)])
    src2 = jnp.concatenate([2 * srcp, 2 * srcp + 1])
    dstp_seg = jnp.concatenate(
        [dst, N + (jnp.arange(pad, dtype=jnp.int32) % 16)])
    dstp_g = jnp.concatenate([dst, jnp.zeros((pad,), jnp.int32)])

    row = lambda v: v.reshape(1, -1)
    w1a, w1b, w1c = W1[:H], W1[H:2 * H], W1[2 * H:]

    cnt = _degree_cnt(dstp_seg)

    h0, hl0 = _node_enc(x, Wn, row(bn), row(gn), row(betan), Wl0)
    s20 = _segsum1(hl0.reshape(2 * N, 128), src2, dstp_seg)
    ehc = _edge_enc(edge_attr, We, row(be), row(ge), row(betae), w1c)
    h1, hl1 = _upd(h0, s20, cnt, Wr0, row(bl0), row(g0), row(beta0),
                   Wl1[:, :128], Wl1[:, 128:], True)
    s21 = _segsum2(hl1.reshape(2 * N, 128), src2, dstp_seg)
    h2, ha, hb = _upd(h1, s21, cnt, Wr1, row(bl1), row(g1), row(beta1),
                      w1a, w1b, False)

    ga, gb = _pair_gather(ha, hb, srcp, dstp_g)
    out = _score(ga, gb, ehc, row(b1), W2, row(b2), W3, row(b3))
    return out[:E, 0]
